# Initial kernel scaffold; baseline (speedup 1.0000x reference)
import jax
import jax.numpy as jnp
from jax.experimental import pallas as pl


def kernel(x, edge_index, batch, W1, b1, W2, b2, W3, b3, fW1, fb1, fW2, fb2, fW3, fb3):
    raise NotImplementedError("write your pallas kernel here")


# speedup vs baseline: 16.9423x; 16.9423x over previous
import dataclasses
import functools

import jax
import jax.numpy as jnp
from jax import lax
from jax.experimental import pallas as pl
from jax.experimental.pallas import tpu as pltpu
from jax.experimental.pallas import tpu_sc as plsc

NC = 2
NS = 16
NW = NC * NS
L = 16

NEG_INF = float("-inf")

_SC_PARAMS = pltpu.CompilerParams()
if "needs_layout_passes" in pltpu.CompilerParams.__dataclass_fields__:
    _SC_PARAMS = dataclasses.replace(_SC_PARAMS, needs_layout_passes=False)


def _worker(c, s):
    return c * NS + s


def _sc_degree(dst3, n, d):
    nw, ch, k = dst3.shape
    rpt = n // NS
    nz = rpt // k
    mesh = plsc.VectorSubcoreMesh(core_axis_name="c", subcore_axis_name="s", num_cores=NC, num_subcores=NS)

    @functools.partial(
        pl.kernel,
        out_type=jax.ShapeDtypeStruct((NC, NS, rpt, d), jnp.float32),
        mesh=mesh,
        compiler_params=_SC_PARAMS,
        scratch_types=[
            pltpu.VMEM_SHARED((n, d), jnp.float32),
            pltpu.VMEM((ch, k), jnp.int32),
            pltpu.VMEM((k, d), jnp.float32),
        ],
    )
    def deg_kernel(dst_hbm, out_hbm, acc, idx_v, buf_v):
        c = lax.axis_index("c")
        s = lax.axis_index("s")
        base = s * rpt

        @pl.loop(0, k)
        def _(r):
            @pl.loop(0, d // L)
            def _(q):
                buf_v[r, pl.ds(q * L, L)] = jnp.zeros((L,), jnp.float32)

        for i in range(nz):
            pltpu.sync_copy(buf_v, acc.at[pl.ds(base + i * k, k)])

        @pl.loop(0, k)
        def _(r):
            @pl.loop(0, d // L)
            def _(q):
                buf_v[r, pl.ds(q * L, L)] = jnp.ones((L,), jnp.float32)

        w = _worker(c, s)
        pltpu.sync_copy(dst_hbm.at[w], idx_v)
        plsc.subcore_barrier()

        @pl.loop(0, ch)
        def _(j):
            pltpu.sync_copy(buf_v, acc.at[idx_v.at[j]], add=True)

        plsc.subcore_barrier()
        pltpu.sync_copy(acc.at[pl.ds(base, rpt)], out_hbm.at[c, s])

    return deg_kernel(dst3)


def _sc_scatter(y, src3, dst3):
    n, d = y.shape
    nw, ch, k = src3.shape
    rpt = n // NS
    nz = rpt // k
    mesh = plsc.VectorSubcoreMesh(core_axis_name="c", subcore_axis_name="s", num_cores=NC, num_subcores=NS)

    @functools.partial(
        pl.kernel,
        out_type=jax.ShapeDtypeStruct((NC, NS, rpt, d), jnp.float32),
        mesh=mesh,
        compiler_params=_SC_PARAMS,
        scratch_types=[
            pltpu.VMEM_SHARED((n, d), jnp.float32),
            pltpu.VMEM((ch, k), jnp.int32),
            pltpu.VMEM((ch, k), jnp.int32),
            pltpu.VMEM((k, d), jnp.float32),
        ],
    )
    def scatter_kernel(y_hbm, src_hbm, dst_hbm, out_hbm, acc, src_v, dst_v, rows_v):
        c = lax.axis_index("c")
        s = lax.axis_index("s")
        w = _worker(c, s)
        base = s * rpt

        @pl.loop(0, k)
        def _(r):
            @pl.loop(0, d // L)
            def _(q):
                rows_v[r, pl.ds(q * L, L)] = jnp.zeros((L,), jnp.float32)

        for i in range(nz):
            pltpu.sync_copy(rows_v, acc.at[pl.ds(base + i * k, k)])

        pltpu.sync_copy(src_hbm.at[w], src_v)
        pltpu.sync_copy(dst_hbm.at[w], dst_v)
        plsc.subcore_barrier()

        @pl.loop(0, ch)
        def _(j):
            pltpu.sync_copy(y_hbm.at[src_v.at[j]], rows_v)
            pltpu.sync_copy(rows_v, acc.at[dst_v.at[j]], add=True)

        plsc.subcore_barrier()
        pltpu.sync_copy(acc.at[pl.ds(base, rpt)], out_hbm.at[c, s])

    return scatter_kernel(y, src3, dst3)


def _sc_segmax(h, batch, g):
    n, d = h.shape
    rb = 32
    mesh = plsc.VectorSubcoreMesh(core_axis_name="c", subcore_axis_name="s", num_cores=NC, num_subcores=NS)

    @functools.partial(
        pl.kernel,
        out_type=jax.ShapeDtypeStruct((NW, 2, d), jnp.float32),
        mesh=mesh,
        compiler_params=_SC_PARAMS,
        scratch_types=[
            pltpu.VMEM((n,), jnp.int32),
            pltpu.VMEM((rb, d), jnp.float32),
            pltpu.VMEM((2, d), jnp.float32),
        ],
    )
    def segmax_kernel(h_hbm, batch_hbm, out_hbm, batch_v, buf_v, acc_v):
        c = lax.axis_index("c")
        s = lax.axis_index("s")
        w = _worker(c, s)
        g0 = 2 * w

        pltpu.sync_copy(batch_hbm, batch_v)

        zeros = jnp.zeros((L,), jnp.int32)

        def count_body(i, carry):
            c0, c1, c2 = carry
            b = batch_v[pl.ds(i * L, L)]
            c0 = c0 + (b < g0).astype(jnp.int32)
            c1 = c1 + (b < g0 + 1).astype(jnp.int32)
            c2 = c2 + (b < g0 + 2).astype(jnp.int32)
            return c0, c1, c2

        c0, c1, c2 = lax.fori_loop(0, n // L, count_body, (zeros, zeros, zeros))
        s0 = jnp.sum(c0)
        s1 = jnp.sum(c1)
        s2 = jnp.sum(c2)

        for gl in range(2):
            for q in range(d // L):
                acc_v[gl, pl.ds(q * L, L)] = jnp.full((L,), NEG_INF, jnp.float32)

        for gl, gs, ge in ((0, s0, s1), (1, s1, s2)):
            gsa = (gs // 8) * 8
            nchunk = (ge - gsa + rb - 1) // rb

            @pl.loop(0, nchunk)
            def _(t):
                r0 = jnp.minimum(gsa + t * rb, n - rb)
                pltpu.sync_copy(h_hbm.at[pl.ds(r0, rb)], buf_v)
                for kk in range(rb):
                    valid = jnp.logical_and(r0 + kk >= gs, r0 + kk < ge)
                    for q in range(d // L):
                        sl = pl.ds(q * L, L)
                        v = jnp.where(valid, buf_v[kk, sl],
                                      jnp.full((L,), NEG_INF, jnp.float32))
                        acc_v[gl, sl] = jnp.maximum(acc_v[gl, sl], v)

        pltpu.sync_copy(acc_v, out_hbm.at[w])

    return segmax_kernel(h, batch)


_TC_BLOCK = 1000


def _tc_prep(deg2, x, w1):
    n, d = x.shape
    b = _TC_BLOCK

    def body(deg_ref, x_ref, w_ref, y_ref, dinv_ref):
        deg = deg_ref[0, :, 0:1] + deg_ref[1, :, 0:1] + 1.0
        dinv = lax.rsqrt(deg)
        dinv_ref[...] = dinv
        y_ref[...] = dinv * jnp.dot(x_ref[...], w_ref[...],
                                    preferred_element_type=jnp.float32)

    return pl.pallas_call(
        body,
        grid=(n // b,),
        in_specs=[
            pl.BlockSpec((NC, b, d), lambda i: (0, i, 0)),
            pl.BlockSpec((b, d), lambda i: (i, 0)),
            pl.BlockSpec((d, d), lambda i: (0, 0)),
        ],
        out_specs=[
            pl.BlockSpec((b, d), lambda i: (i, 0)),
            pl.BlockSpec((b, 1), lambda i: (i, 0)),
        ],
        out_shape=[
            jax.ShapeDtypeStruct((n, d), jnp.float32),
            jax.ShapeDtypeStruct((n, 1), jnp.float32),
        ],
    )(deg2, x, w1)


def _tc_mid(y, z, dinv, bias, w_next):
    n, d = y.shape
    b = _TC_BLOCK

    def body(y_ref, z_ref, dinv_ref, b_ref, w_ref, out_ref):
        zs = y_ref[...] + z_ref[0] + z_ref[1]
        h = jnp.maximum(dinv_ref[...] * zs + b_ref[...], 0.0)
        out_ref[...] = dinv_ref[...] * jnp.dot(h, w_ref[...],
                                               preferred_element_type=jnp.float32)

    return pl.pallas_call(
        body,
        grid=(n // b,),
        in_specs=[
            pl.BlockSpec((b, d), lambda i: (i, 0)),
            pl.BlockSpec((NC, b, d), lambda i: (0, i, 0)),
            pl.BlockSpec((b, 1), lambda i: (i, 0)),
            pl.BlockSpec((1, d), lambda i: (0, 0)),
            pl.BlockSpec((d, d), lambda i: (0, 0)),
        ],
        out_specs=pl.BlockSpec((b, d), lambda i: (i, 0)),
        out_shape=jax.ShapeDtypeStruct((n, d), jnp.float32),
    )(y, z, dinv, bias, w_next)


def _tc_h3(y, z, dinv, bias):
    n, d = y.shape
    b = _TC_BLOCK

    def body(y_ref, z_ref, dinv_ref, b_ref, out_ref):
        zs = y_ref[...] + z_ref[0] + z_ref[1]
        out_ref[...] = jnp.maximum(dinv_ref[...] * zs + b_ref[...], 0.0)

    return pl.pallas_call(
        body,
        grid=(n // b,),
        in_specs=[
            pl.BlockSpec((b, d), lambda i: (i, 0)),
            pl.BlockSpec((NC, b, d), lambda i: (0, i, 0)),
            pl.BlockSpec((b, 1), lambda i: (i, 0)),
            pl.BlockSpec((1, d), lambda i: (0, 0)),
        ],
        out_specs=pl.BlockSpec((b, d), lambda i: (i, 0)),
        out_shape=jax.ShapeDtypeStruct((n, d), jnp.float32),
    )(y, z, dinv, bias)


def _tc_head(g, fw1, fb1, fw2, fb2, fw3, fb3):
    gg, d = g.shape

    def body(g_ref, w1_ref, b1_ref, w2_ref, b2_ref, w3_ref, b3_ref, out_ref):
        h = jnp.maximum(jnp.dot(g_ref[...], w1_ref[...],
                                preferred_element_type=jnp.float32) + b1_ref[...], 0.0)
        h = jnp.maximum(jnp.dot(h, w2_ref[...],
                                preferred_element_type=jnp.float32) + b2_ref[...], 0.0)
        out_ref[...] = jnp.dot(h, w3_ref[...],
                               preferred_element_type=jnp.float32) + b3_ref[...]

    return pl.pallas_call(
        body,
        out_shape=jax.ShapeDtypeStruct((gg, fw3.shape[1]), jnp.float32),
    )(g, fw1, fb1, fw2, fb2, fw3, fb3)


def kernel(x, edge_index, batch, W1, b1, W2, b2, W3, b3,
           fW1, fb1, fW2, fb2, fW3, fb3):
    n, d = x.shape
    e = edge_index.shape[1]
    g = 2 * NW

    assert e % NW == 0
    ew = e // NW
    k = 125
    assert ew % k == 0 and (n // NS) % k == 0 and n % NS == 0

    src3 = edge_index[0].reshape(NW, ew // k, k)
    dst3 = edge_index[1].reshape(NW, ew // k, k)

    deg2 = _sc_degree(dst3, n, d).reshape(NC, n, d)
    y1, dinv = _tc_prep(deg2, x, W1)
    z1 = _sc_scatter(y1, src3, dst3).reshape(NC, n, d)
    y2 = _tc_mid(y1, z1, dinv, b1.reshape(1, d), W2)
    z2 = _sc_scatter(y2, src3, dst3).reshape(NC, n, d)
    y3 = _tc_mid(y2, z2, dinv, b2.reshape(1, d), W3)
    z3 = _sc_scatter(y3, src3, dst3).reshape(NC, n, d)
    h3 = _tc_h3(y3, z3, dinv, b3.reshape(1, d))
    gmax = _sc_segmax(h3, batch, g).reshape(g, d)
    return _tc_head(gmax, fW1, fb1.reshape(1, -1), fW2, fb2.reshape(1, -1),
                    fW3, fb3.reshape(1, -1))


# speedup vs baseline: 19.6757x; 1.1613x over previous
import dataclasses
import functools

import jax
import jax.numpy as jnp
from jax import lax
from jax.experimental import pallas as pl
from jax.experimental.pallas import tpu as pltpu
from jax.experimental.pallas import tpu_sc as plsc

NC = 2
NS = 16
NW = NC * NS
L = 16

NEG_INF = float("-inf")

_SC_PARAMS = pltpu.CompilerParams()
if "needs_layout_passes" in pltpu.CompilerParams.__dataclass_fields__:
    _SC_PARAMS = dataclasses.replace(_SC_PARAMS, needs_layout_passes=False)


def _worker(c, s):
    return c * NS + s


def _sc_degree(dst3, n, d):
    nw, ch, k = dst3.shape
    rpt = n // NS
    nzf, rem = divmod(rpt, k)
    mesh = plsc.VectorSubcoreMesh(core_axis_name="c", subcore_axis_name="s", num_cores=NC, num_subcores=NS)

    @functools.partial(
        pl.kernel,
        out_type=jax.ShapeDtypeStruct((NC, NS, rpt, d), jnp.float32),
        mesh=mesh,
        compiler_params=_SC_PARAMS,
        scratch_types=[
            pltpu.VMEM_SHARED((n, d), jnp.float32),
            pltpu.VMEM((ch, k), jnp.int32),
            pltpu.VMEM((k, d), jnp.float32),
            pltpu.SemaphoreType.DMA,
        ],
    )
    def deg_kernel(dst_hbm, out_hbm, acc, idx_v, buf_v, sem):
        c = lax.axis_index("c")
        s = lax.axis_index("s")
        base = s * rpt

        @pl.loop(0, k)
        def _(r):
            @pl.loop(0, d // L)
            def _(q):
                buf_v[r, pl.ds(q * L, L)] = jnp.zeros((L,), jnp.float32)

        for i in range(nzf):
            pltpu.sync_copy(buf_v, acc.at[pl.ds(base + i * k, k)])
        if rem:
            pltpu.sync_copy(buf_v.at[pl.ds(0, rem)],
                            acc.at[pl.ds(base + nzf * k, rem)])

        @pl.loop(0, k)
        def _(r):
            @pl.loop(0, d // L)
            def _(q):
                buf_v[r, pl.ds(q * L, L)] = jnp.ones((L,), jnp.float32)

        w = _worker(c, s)
        pltpu.sync_copy(dst_hbm.at[w], idx_v)
        plsc.subcore_barrier()

        gk = 8

        @pl.loop(0, ch // gk)
        def _(t):
            for i in range(gk):
                pltpu.async_copy(buf_v, acc.at[idx_v.at[t * gk + i]], sem,
                                 add=True)
            for i in range(gk):
                pltpu.make_async_copy(buf_v, acc.at[idx_v.at[t * gk + i]],
                                      sem).wait()

        for j in range((ch // gk) * gk, ch):
            pltpu.async_copy(buf_v, acc.at[idx_v.at[j]], sem, add=True)
        for j in range((ch // gk) * gk, ch):
            pltpu.make_async_copy(buf_v, acc.at[idx_v.at[j]], sem).wait()

        plsc.subcore_barrier()
        pltpu.sync_copy(acc.at[pl.ds(base, rpt)], out_hbm.at[c, s])

    return deg_kernel(dst3)


def _sc_scatter(y, src3, dst3, n_acc):
    _, d = y.shape
    nw, ch, k = dst3.shape
    rpt = n_acc // NS
    mesh = plsc.VectorSubcoreMesh(core_axis_name="c", subcore_axis_name="s", num_cores=NC, num_subcores=NS)

    nzf, rem = divmod(rpt, k)
    assert ch % 2 == 0 and ch >= 4

    @functools.partial(
        pl.kernel,
        out_type=jax.ShapeDtypeStruct((NC, NS, rpt, d), jnp.float32),
        mesh=mesh,
        compiler_params=_SC_PARAMS,
        scratch_types=[
            pltpu.VMEM_SHARED((n_acc, d), jnp.float32),
            pltpu.VMEM((ch * k,), jnp.int32),
            pltpu.VMEM((ch, k), jnp.int32),
            [pltpu.VMEM((k, d), jnp.float32) for _ in range(2)],
            [pltpu.SemaphoreType.DMA for _ in range(2)],
            [pltpu.SemaphoreType.DMA for _ in range(2)],
        ],
    )
    def scatter_kernel(y_hbm, src_hbm, dst_hbm, out_hbm, acc, src_v, dst_v,
                       bufs, gsems, ssems):
        c = lax.axis_index("c")
        s = lax.axis_index("s")
        w = _worker(c, s)
        base = s * rpt

        @pl.loop(0, k)
        def _(r):
            @pl.loop(0, d // L)
            def _(q):
                bufs[0][r, pl.ds(q * L, L)] = jnp.zeros((L,), jnp.float32)

        for i in range(nzf):
            pltpu.sync_copy(bufs[0], acc.at[pl.ds(base + i * k, k)])
        if rem:
            pltpu.sync_copy(bufs[0].at[pl.ds(0, rem)],
                            acc.at[pl.ds(base + nzf * k, rem)])

        pltpu.sync_copy(src_hbm.at[w], src_v)
        pltpu.sync_copy(dst_hbm.at[w], dst_v)
        plsc.subcore_barrier()

        pltpu.async_copy(y_hbm.at[src_v.at[pl.ds(0, k)]], bufs[0], gsems[0])

        @pl.loop(0, ch // 2)
        def _(t):
            for i in range(2):
                jj = t * 2 + i
                bo = 1 - i
                pltpu.make_async_copy(y_hbm.at[src_v.at[pl.ds(jj * k, k)]],
                                      bufs[i], gsems[i]).wait()
                pltpu.async_copy(bufs[i], acc.at[dst_v.at[jj]], ssems[i],
                                 add=True)

                @pl.when(jnp.logical_and(jj >= 1, jj + 1 < ch))
                def _():
                    pltpu.make_async_copy(bufs[bo], acc.at[dst_v.at[jj - 1]],
                                          ssems[bo]).wait()

                @pl.when(jj + 1 < ch)
                def _():
                    pltpu.async_copy(y_hbm.at[src_v.at[pl.ds((jj + 1) * k, k)]],
                                     bufs[bo], gsems[bo])

        pltpu.make_async_copy(bufs[0], acc.at[dst_v.at[ch - 2]], ssems[0]).wait()
        pltpu.make_async_copy(bufs[1], acc.at[dst_v.at[ch - 1]], ssems[1]).wait()

        plsc.subcore_barrier()
        pltpu.sync_copy(acc.at[pl.ds(base, rpt)], out_hbm.at[c, s])

    return scatter_kernel(y, src3, dst3)


def _sc_segmax(h, batch, g):
    n, d = h.shape
    rb = 32
    mesh = plsc.VectorSubcoreMesh(core_axis_name="c", subcore_axis_name="s", num_cores=NC, num_subcores=NS)

    @functools.partial(
        pl.kernel,
        out_type=jax.ShapeDtypeStruct((NW, 2, d), jnp.float32),
        mesh=mesh,
        compiler_params=_SC_PARAMS,
        scratch_types=[
            pltpu.VMEM((n,), jnp.int32),
            pltpu.VMEM((rb, d), jnp.float32),
            pltpu.VMEM((2, d), jnp.float32),
        ],
    )
    def segmax_kernel(h_hbm, batch_hbm, out_hbm, batch_v, buf_v, acc_v):
        c = lax.axis_index("c")
        s = lax.axis_index("s")
        w = _worker(c, s)
        g0 = 2 * w

        pltpu.sync_copy(batch_hbm, batch_v)

        zeros = jnp.zeros((L,), jnp.int32)

        def count_body(i, carry):
            c0, c1, c2 = carry
            b = batch_v[pl.ds(i * L, L)]
            c0 = c0 + (b < g0).astype(jnp.int32)
            c1 = c1 + (b < g0 + 1).astype(jnp.int32)
            c2 = c2 + (b < g0 + 2).astype(jnp.int32)
            return c0, c1, c2

        c0, c1, c2 = lax.fori_loop(0, n // L, count_body, (zeros, zeros, zeros))
        s0 = jnp.sum(c0)
        s1 = jnp.sum(c1)
        s2 = jnp.sum(c2)

        for gl in range(2):
            for q in range(d // L):
                acc_v[gl, pl.ds(q * L, L)] = jnp.full((L,), NEG_INF, jnp.float32)

        for gl, gs, ge in ((0, s0, s1), (1, s1, s2)):
            gsa = (gs // 8) * 8
            nchunk = (ge - gsa + rb - 1) // rb

            @pl.loop(0, nchunk)
            def _(t):
                r0 = jnp.minimum(gsa + t * rb, n - rb)
                pltpu.sync_copy(h_hbm.at[pl.ds(r0, rb)], buf_v)
                for kk in range(rb):
                    valid = jnp.logical_and(r0 + kk >= gs, r0 + kk < ge)
                    for q in range(d // L):
                        sl = pl.ds(q * L, L)
                        v = jnp.where(valid, buf_v[kk, sl],
                                      jnp.full((L,), NEG_INF, jnp.float32))
                        acc_v[gl, sl] = jnp.maximum(acc_v[gl, sl], v)

        pltpu.sync_copy(acc_v, out_hbm.at[w])

    return segmax_kernel(h, batch)


_TC_BLOCK = 1000


def _tc_prep(deg2, x, w1):
    n, d = x.shape
    b = _TC_BLOCK

    def body(deg_ref, x_ref, w_ref, y_ref, dinv_ref):
        deg = deg_ref[0, :, 0:1] + deg_ref[1, :, 0:1] + 1.0
        dinv = lax.rsqrt(deg)
        dinv_ref[...] = dinv
        y_ref[...] = dinv * jnp.dot(x_ref[...], w_ref[...],
                                    preferred_element_type=jnp.float32)

    return pl.pallas_call(
        body,
        grid=(n // b,),
        in_specs=[
            pl.BlockSpec((NC, b, d), lambda i: (0, i, 0)),
            pl.BlockSpec((b, d), lambda i: (i, 0)),
            pl.BlockSpec((d, d), lambda i: (0, 0)),
        ],
        out_specs=[
            pl.BlockSpec((b, d), lambda i: (i, 0)),
            pl.BlockSpec((b, 1), lambda i: (i, 0)),
        ],
        out_shape=[
            jax.ShapeDtypeStruct((n, d), jnp.float32),
            jax.ShapeDtypeStruct((n, 1), jnp.float32),
        ],
    )(deg2, x, w1)


def _tc_mid(y, z, dinv, bias, w_next):
    n, d = y.shape
    b = _TC_BLOCK

    def body(y_ref, z_ref, dinv_ref, b_ref, w_ref, out_ref):
        zs = y_ref[...] + z_ref[0] + z_ref[1]
        h = jnp.maximum(dinv_ref[...] * zs + b_ref[...], 0.0)
        out_ref[...] = dinv_ref[...] * jnp.dot(h, w_ref[...],
                                               preferred_element_type=jnp.float32)

    return pl.pallas_call(
        body,
        grid=(n // b,),
        in_specs=[
            pl.BlockSpec((b, d), lambda i: (i, 0)),
            pl.BlockSpec((NC, b, d), lambda i: (0, i, 0)),
            pl.BlockSpec((b, 1), lambda i: (i, 0)),
            pl.BlockSpec((1, d), lambda i: (0, 0)),
            pl.BlockSpec((d, d), lambda i: (0, 0)),
        ],
        out_specs=pl.BlockSpec((b, d), lambda i: (i, 0)),
        out_shape=jax.ShapeDtypeStruct((n, d), jnp.float32),
    )(y, z, dinv, bias, w_next)


def _tc_h3(y, z, dinv, bias):
    n, d = y.shape
    b = _TC_BLOCK

    def body(y_ref, z_ref, dinv_ref, b_ref, out_ref):
        zs = y_ref[...] + z_ref[0] + z_ref[1]
        out_ref[...] = jnp.maximum(dinv_ref[...] * zs + b_ref[...], 0.0)

    return pl.pallas_call(
        body,
        grid=(n // b,),
        in_specs=[
            pl.BlockSpec((b, d), lambda i: (i, 0)),
            pl.BlockSpec((NC, b, d), lambda i: (0, i, 0)),
            pl.BlockSpec((b, 1), lambda i: (i, 0)),
            pl.BlockSpec((1, d), lambda i: (0, 0)),
        ],
        out_specs=pl.BlockSpec((b, d), lambda i: (i, 0)),
        out_shape=jax.ShapeDtypeStruct((n, d), jnp.float32),
    )(y, z, dinv, bias)


def _tc_head(g, fw1, fb1, fw2, fb2, fw3, fb3):
    gg, d = g.shape

    def body(g_ref, w1_ref, b1_ref, w2_ref, b2_ref, w3_ref, b3_ref, out_ref):
        h = jnp.maximum(jnp.dot(g_ref[...], w1_ref[...],
                                preferred_element_type=jnp.float32) + b1_ref[...], 0.0)
        h = jnp.maximum(jnp.dot(h, w2_ref[...],
                                preferred_element_type=jnp.float32) + b2_ref[...], 0.0)
        out_ref[...] = jnp.dot(h, w3_ref[...],
                               preferred_element_type=jnp.float32) + b3_ref[...]

    return pl.pallas_call(
        body,
        out_shape=jax.ShapeDtypeStruct((gg, fw3.shape[1]), jnp.float32),
    )(g, fw1, fb1, fw2, fb2, fw3, fb3)


def kernel(x, edge_index, batch, W1, b1, W2, b2, W3, b3,
           fW1, fb1, fW2, fb2, fW3, fb3):
    n, d = x.shape
    e = edge_index.shape[1]
    g = 2 * NW

    k = 112
    n1 = n + 16
    e1 = -(-e // (2 * NW * k)) * 2 * NW * k
    pad = e1 - e
    ew = e1 // NW
    ch = ew // k
    assert ch % 2 == 0 and n % NS == 0 and n1 % NS == 0 and k % 8 == 0

    pidx = jnp.arange(pad, dtype=jnp.int32)
    src_p = jnp.concatenate([edge_index[0], n + (pidx % 8)])
    dst_conv = jnp.concatenate([edge_index[1], pidx % 1024])
    dst_deg = jnp.concatenate([edge_index[1], n + (pidx % 16)])
    src3 = src_p.reshape(NW, ch * k)
    dst3 = dst_conv.reshape(NW, ch, k)
    dst3d = dst_deg.reshape(NW, ch, k)
    zrows = jnp.zeros((8, d), jnp.float32)

    deg2 = _sc_degree(dst3d, n1, d).reshape(NC, n1, d)
    y1, dinv = _tc_prep(deg2, x, W1)
    z1 = _sc_scatter(jnp.concatenate([y1, zrows]), src3, dst3, n).reshape(NC, n, d)
    y2 = _tc_mid(y1, z1, dinv, b1.reshape(1, d), W2)
    z2 = _sc_scatter(jnp.concatenate([y2, zrows]), src3, dst3, n).reshape(NC, n, d)
    y3 = _tc_mid(y2, z2, dinv, b2.reshape(1, d), W3)
    z3 = _sc_scatter(jnp.concatenate([y3, zrows]), src3, dst3, n).reshape(NC, n, d)
    h3 = _tc_h3(y3, z3, dinv, b3.reshape(1, d))
    gmax = _sc_segmax(h3, batch, g).reshape(g, d)
    return _tc_head(gmax, fW1, fb1.reshape(1, -1), fW2, fb2.reshape(1, -1),
                    fW3, fb3.reshape(1, -1))
